# baseline (speedup 1.0000x reference)
import functools

import jax
import jax.numpy as jnp
from jax import lax
from jax.experimental import pallas as pl
from jax.experimental.pallas import tpu as pltpu

_C0, _C1, _C2 = 0.04, 5.0, 140.0
_THRESH = 30.0
_V_RESET = -65.0


def _snn_body(x_ref, p_ref, w_ref, out_ref, *, num_layers, steps, unroll):
    L = num_layers
    N = x_ref.shape[2]

    a_l = [p_ref[0, i] for i in range(L)]
    b_l = [p_ref[1, i] for i in range(L)]
    c_l = [p_ref[2, i] for i in range(L)]
    d_l = [p_ref[3, i] for i in range(L)]
    bias140_l = [p_ref[4, i] + _C2 for i in range(L)]

    v0 = tuple(jnp.full_like(b_l[i], _V_RESET) for i in range(L))
    u0 = tuple(b_l[i] * _V_RESET for i in range(L))

    def matvec(layer, s):
        w2 = w_ref[layer]
        acc0 = s[:, 0:1] * w2[0]
        acc1 = s[:, 1:2] * w2[1]
        for n in range(2, N, 2):
            acc0 = acc0 + s[:, n:n + 1] * w2[n]
            acc1 = acc1 + s[:, n + 1:n + 2] * w2[n + 1]
        return acc0 + acc1

    def step(t, carry):
        v_st, u_st = carry
        v_st, u_st = list(v_st), list(u_st)
        s = x_ref[t]
        for i in range(L):
            if i > 0:
                s = matvec(i - 1, s)
            vi, ui = v_st[i], u_st[i]
            dv = (_C0 * vi + _C1) * vi + bias140_l[i] - ui + s
            du = a_l[i] * (b_l[i] * vi - ui)
            v_new = vi + dv
            u_new = ui + du
            spiked = v_new > _THRESH
            s = spiked.astype(jnp.float32)
            v_st[i] = jnp.where(spiked, c_l[i], v_new)
            u_st[i] = jnp.where(spiked, u_new + d_l[i], u_new)
        out_ref[t] = s
        return tuple(v_st), tuple(u_st)

    lax.fori_loop(0, steps, step, (v0, u0), unroll=unroll)


def _snn_forward(x, a, b, c, d, bias, w, *, steps, unroll):
    T, B, N = x.shape
    L = a.shape[0]
    x = x[:steps]

    params = jnp.stack([a, b, c, d, bias], axis=0)
    w2 = jnp.transpose(w, (0, 2, 1, 3))

    body = functools.partial(_snn_body, num_layers=L, steps=steps,
                             unroll=unroll)

    out = pl.pallas_call(
        body,
        out_shape=jax.ShapeDtypeStruct((steps, B, N), jnp.float32),
        grid_spec=pltpu.PrefetchScalarGridSpec(
            num_scalar_prefetch=0,
            grid=(1,),
            in_specs=[
                pl.BlockSpec((steps, B, N), lambda g: (0, 0, 0)),
                pl.BlockSpec((5, L, B, N), lambda g: (0, 0, 0, 0)),
                pl.BlockSpec((L - 1, N, B, N), lambda g: (0, 0, 0, 0)),
            ],
            out_specs=pl.BlockSpec((steps, B, N), lambda g: (0, 0, 0)),
        ),
        compiler_params=pltpu.CompilerParams(
            dimension_semantics=("arbitrary",),
            vmem_limit_bytes=64 * 1024 * 1024,
        ),
    )(x, params, w2)
    return out


def kernel(x, a, b, c, d, bias, w):
    return _snn_forward(x, a, b, c, d, bias, w, steps=320, unroll=8)


# speedup vs baseline: 1.0581x; 1.0581x over previous
import functools

import jax
import jax.numpy as jnp
from jax import lax
from jax.experimental import pallas as pl
from jax.experimental.pallas import tpu as pltpu

_C0, _C1 = 0.04, 5.0
_THRESH = 30.0
_V_RESET = -65.0


def _snn_body_t(x_ref, p_ref, w_ref, out_ref, *, num_layers, steps, unroll):
    L = num_layers
    N2 = x_ref.shape[1]
    B2 = x_ref.shape[2]
    H = B2 // 2

    lane = lax.broadcasted_iota(jnp.int32, (N2, B2), 1)
    low_half = lane < H

    v0 = tuple(jnp.full((N2, B2), _V_RESET, jnp.float32) for _ in range(L))
    u0 = tuple(p_ref[1, i] * _V_RESET for i in range(L))

    def matvec(layer, s):
        acc0 = s[0:1, :] * w_ref[layer, 0]
        acc1 = s[1:2, :] * w_ref[layer, 1]
        for n2 in range(2, N2, 2):
            acc0 = acc0 + s[n2:n2 + 1, :] * w_ref[layer, n2]
            acc1 = acc1 + s[n2 + 1:n2 + 2, :] * w_ref[layer, n2 + 1]
        acc = acc0 + acc1
        f = acc + jnp.concatenate([acc[:, H:], acc[:, :H]], axis=1)
        return jnp.where(low_half, f[0:N2, :], f[N2:2 * N2, :])

    def step(t, carry):
        v_st, u_st = carry
        v_st, u_st = list(v_st), list(u_st)
        s = x_ref[t]
        for i in range(L):
            if i > 0:
                s = matvec(i - 1, s)
            vi, ui = v_st[i], u_st[i]
            dv = (_C0 * vi + _C1) * vi + p_ref[4, i] - ui + s
            du = p_ref[0, i] * (p_ref[1, i] * vi - ui)
            v_new = vi + dv
            u_new = ui + du
            spiked = v_new > _THRESH
            s = spiked.astype(jnp.float32)
            v_st[i] = jnp.where(spiked, p_ref[2, i], v_new)
            u_st[i] = jnp.where(spiked, u_new + p_ref[3, i], u_new)
        out_ref[t] = s
        return tuple(v_st), tuple(u_st)

    lax.fori_loop(0, steps, step, (v0, u0), unroll=unroll)


def _pack(arr):
    B, N = arr.shape[-2], arr.shape[-1]
    lead = arr.shape[:-2]
    perm = list(range(arr.ndim))
    perm[-2], perm[-1] = perm[-1], perm[-2]
    t = jnp.transpose(arr, perm)
    t = t.reshape(*lead, 2, N // 2, B)
    t = jnp.moveaxis(t, -3, -2)
    return t.reshape(*lead, N // 2, 2 * B)


def _snn_forward_t(x, a, b, c, d, bias, w, *, steps, unroll):
    T, B, N = x.shape
    L = a.shape[0]
    x = x[:steps]

    xp = _pack(x)
    params = _pack(jnp.stack([a, b, c, d, bias + 140.0], axis=0))
    wt = jnp.transpose(w, (0, 2, 3, 1))
    w4 = wt.reshape(L - 1, 2, N // 2, N, B).transpose(0, 2, 3, 1, 4)
    w4 = w4.reshape(L - 1, N // 2, N, 2 * B)

    body = functools.partial(_snn_body_t, num_layers=L, steps=steps,
                             unroll=unroll)

    outp = pl.pallas_call(
        body,
        out_shape=jax.ShapeDtypeStruct((steps, N // 2, 2 * B), jnp.float32),
        grid_spec=pltpu.PrefetchScalarGridSpec(
            num_scalar_prefetch=0,
            grid=(1,),
            in_specs=[
                pl.BlockSpec((steps, N // 2, 2 * B), lambda g: (0, 0, 0)),
                pl.BlockSpec((5, L, N // 2, 2 * B), lambda g: (0, 0, 0, 0)),
                pl.BlockSpec((L - 1, N // 2, N, 2 * B),
                             lambda g: (0, 0, 0, 0)),
            ],
            out_specs=pl.BlockSpec((steps, N // 2, 2 * B),
                                   lambda g: (0, 0, 0)),
        ),
        compiler_params=pltpu.CompilerParams(
            dimension_semantics=("arbitrary",),
            vmem_limit_bytes=64 * 1024 * 1024,
        ),
    )(xp, params, w4)
    out = outp.reshape(steps, N // 2, 2, B).transpose(0, 3, 2, 1)
    return out.reshape(steps, B, N)


def kernel(x, a, b, c, d, bias, w):
    return _snn_forward_t(x, a, b, c, d, bias, w, steps=320, unroll=8)


# speedup vs baseline: 1.4299x; 1.3514x over previous
import functools

import jax
import jax.numpy as jnp
from jax import lax
from jax.experimental import pallas as pl
from jax.experimental.pallas import tpu as pltpu

_C0, _C1 = 0.04, 5.0
_THRESH = 30.0
_V_RESET = -65.0


def _snn_body_t(x_ref, p_ref, w_ref, out_ref, s_ref, *,
                num_layers, steps, unroll, mv_unroll):
    L = num_layers
    N2 = x_ref.shape[1]
    B2 = x_ref.shape[2]
    H = B2 // 2

    lane = lax.broadcasted_iota(jnp.int32, (N2, B2), 1)
    low_half = lane < H

    v0 = tuple(jnp.full((N2, B2), _V_RESET, jnp.float32) for _ in range(L))
    u0 = tuple(p_ref[1, i] * _V_RESET for i in range(L))

    def matvec(layer):
        def mv_step(n2, carry):
            acc0, acc1 = carry
            r0 = s_ref[2 * n2]
            r1 = s_ref[2 * n2 + 1]
            acc0 = acc0 + r0[None, :] * w_ref[layer, 2 * n2]
            acc1 = acc1 + r1[None, :] * w_ref[layer, 2 * n2 + 1]
            return acc0, acc1

        z = jnp.zeros((N2 * 2, B2), jnp.float32)
        acc0, acc1 = lax.fori_loop(0, N2 // 2, mv_step, (z, z),
                                   unroll=mv_unroll)
        acc = acc0 + acc1
        f = acc + jnp.concatenate([acc[:, H:], acc[:, :H]], axis=1)
        return jnp.where(low_half, f[0:N2, :], f[N2:2 * N2, :])

    def step(t, carry):
        v_st, u_st = carry
        v_st, u_st = list(v_st), list(u_st)
        s = x_ref[t]
        for i in range(L):
            if i > 0:
                s = matvec(i - 1)
            vi, ui = v_st[i], u_st[i]
            dv = (_C0 * vi + _C1) * vi + p_ref[4, i] - ui + s
            du = p_ref[0, i] * (p_ref[1, i] * vi - ui)
            v_new = vi + dv
            u_new = ui + du
            spiked = v_new > _THRESH
            s = spiked.astype(jnp.float32)
            v_st[i] = jnp.where(spiked, p_ref[2, i], v_new)
            u_st[i] = jnp.where(spiked, u_new + p_ref[3, i], u_new)
            if i < L - 1:
                s_ref[...] = s
        out_ref[t] = s
        return tuple(v_st), tuple(u_st)

    lax.fori_loop(0, steps, step, (v0, u0), unroll=unroll)


def _pack(arr):
    B, N = arr.shape[-2], arr.shape[-1]
    lead = arr.shape[:-2]
    perm = list(range(arr.ndim))
    perm[-2], perm[-1] = perm[-1], perm[-2]
    t = jnp.transpose(arr, perm)
    t = t.reshape(*lead, 2, N // 2, B)
    t = jnp.moveaxis(t, -3, -2)
    return t.reshape(*lead, N // 2, 2 * B)


def _snn_forward_t(x, a, b, c, d, bias, w, *, steps, unroll, mv_unroll):
    T, B, N = x.shape
    L = a.shape[0]
    x = x[:steps]

    xp = _pack(x)
    params = _pack(jnp.stack([a, b, c, d, bias + 140.0], axis=0))
    wt = jnp.transpose(w, (0, 2, 3, 1))
    w4 = wt.reshape(L - 1, 2, N // 2, N, B).transpose(0, 2, 3, 1, 4)
    w4 = w4.reshape(L - 1, N // 2, N, 2 * B)

    body = functools.partial(_snn_body_t, num_layers=L, steps=steps,
                             unroll=unroll, mv_unroll=mv_unroll)

    outp = pl.pallas_call(
        body,
        out_shape=jax.ShapeDtypeStruct((steps, N // 2, 2 * B), jnp.float32),
        grid_spec=pltpu.PrefetchScalarGridSpec(
            num_scalar_prefetch=0,
            grid=(1,),
            in_specs=[
                pl.BlockSpec((steps, N // 2, 2 * B), lambda g: (0, 0, 0)),
                pl.BlockSpec((5, L, N // 2, 2 * B), lambda g: (0, 0, 0, 0)),
                pl.BlockSpec((L - 1, N // 2, N, 2 * B),
                             lambda g: (0, 0, 0, 0)),
            ],
            out_specs=pl.BlockSpec((steps, N // 2, 2 * B),
                                   lambda g: (0, 0, 0)),
            scratch_shapes=[
                pltpu.VMEM((N // 2, 2 * B), jnp.float32),
            ],
        ),
        compiler_params=pltpu.CompilerParams(
            dimension_semantics=("arbitrary",),
            vmem_limit_bytes=64 * 1024 * 1024,
        ),
    )(xp, params, w4)
    out = outp.reshape(steps, N // 2, 2, B).transpose(0, 3, 2, 1)
    return out.reshape(steps, B, N)


def kernel(x, a, b, c, d, bias, w):
    return _snn_forward_t(x, a, b, c, d, bias, w, steps=320, unroll=4,
                          mv_unroll=8)


# speedup vs baseline: 1.5083x; 1.0548x over previous
import functools

import jax
import jax.numpy as jnp
from jax import lax
from jax.experimental import pallas as pl
from jax.experimental.pallas import tpu as pltpu

_C0, _C1 = 0.04, 5.0
_THRESH = 30.0
_V_RESET = -65.0


def _snn_body_t(x_ref, p_ref, w_ref, out_ref, s_ref, *,
                num_layers, steps, unroll, mv_unroll):
    L = num_layers
    N2 = x_ref.shape[1]
    B2 = x_ref.shape[2]
    H = B2 // 2

    lane = lax.broadcasted_iota(jnp.int32, (N2, B2), 1)
    low_half = lane < H

    v0 = tuple(jnp.full((N2, B2), _V_RESET, jnp.float32) for _ in range(L))
    u0 = tuple(p_ref[1, i] * _V_RESET for i in range(L))

    def matvec(layer):
        def mv_step(n2, carry):
            acc0, acc1 = carry
            r0 = s_ref[2 * n2]
            r1 = s_ref[2 * n2 + 1]
            acc0 = acc0 + r0[None, :] * w_ref[layer, 2 * n2]
            acc1 = acc1 + r1[None, :] * w_ref[layer, 2 * n2 + 1]
            return acc0, acc1

        z = jnp.zeros((N2 * 2, B2), jnp.float32)
        acc0, acc1 = lax.fori_loop(0, N2 // 2, mv_step, (z, z),
                                   unroll=mv_unroll)
        acc = acc0 + acc1
        f = acc + jnp.concatenate([acc[:, H:], acc[:, :H]], axis=1)
        return jnp.where(low_half, f[0:N2, :], f[N2:2 * N2, :])

    def step(t, carry):
        v_st, u_st = carry
        v_st, u_st = list(v_st), list(u_st)
        s = x_ref[t]
        for i in range(L):
            if i > 0:
                s = matvec(i - 1)
            vi, ui = v_st[i], u_st[i]
            dv = (_C0 * vi + _C1) * vi + p_ref[4, i] - ui + s
            du = p_ref[0, i] * (p_ref[1, i] * vi - ui)
            v_new = vi + dv
            u_new = ui + du
            spiked = v_new > _THRESH
            s = spiked.astype(jnp.float32)
            v_st[i] = jnp.where(spiked, p_ref[2, i], v_new)
            u_st[i] = jnp.where(spiked, u_new + p_ref[3, i], u_new)
            if i < L - 1:
                s_ref[...] = s
        out_ref[t] = s
        return tuple(v_st), tuple(u_st)

    lax.fori_loop(0, steps, step, (v0, u0), unroll=unroll)


def _pack(arr):
    B, N = arr.shape[-2], arr.shape[-1]
    lead = arr.shape[:-2]
    perm = list(range(arr.ndim))
    perm[-2], perm[-1] = perm[-1], perm[-2]
    t = jnp.transpose(arr, perm)
    t = t.reshape(*lead, 2, N // 2, B)
    t = jnp.moveaxis(t, -3, -2)
    return t.reshape(*lead, N // 2, 2 * B)


def _snn_forward_t(x, a, b, c, d, bias, w, *, steps, unroll, mv_unroll):
    T, B, N = x.shape
    L = a.shape[0]
    x = x[:steps]

    xp = _pack(x)
    params = _pack(jnp.stack([a, b, c, d, bias + 140.0], axis=0))
    wt = jnp.transpose(w, (0, 2, 3, 1))
    w4 = wt.reshape(L - 1, 2, N // 2, N, B).transpose(0, 2, 3, 1, 4)
    w4 = w4.reshape(L - 1, N // 2, N, 2 * B)

    body = functools.partial(_snn_body_t, num_layers=L, steps=steps,
                             unroll=unroll, mv_unroll=mv_unroll)

    outp = pl.pallas_call(
        body,
        out_shape=jax.ShapeDtypeStruct((steps, N // 2, 2 * B), jnp.float32),
        grid_spec=pltpu.PrefetchScalarGridSpec(
            num_scalar_prefetch=0,
            grid=(1,),
            in_specs=[
                pl.BlockSpec((steps, N // 2, 2 * B), lambda g: (0, 0, 0)),
                pl.BlockSpec((5, L, N // 2, 2 * B), lambda g: (0, 0, 0, 0)),
                pl.BlockSpec((L - 1, N // 2, N, 2 * B),
                             lambda g: (0, 0, 0, 0)),
            ],
            out_specs=pl.BlockSpec((steps, N // 2, 2 * B),
                                   lambda g: (0, 0, 0)),
            scratch_shapes=[
                pltpu.VMEM((N // 2, 2 * B), jnp.float32),
            ],
        ),
        compiler_params=pltpu.CompilerParams(
            dimension_semantics=("arbitrary",),
            vmem_limit_bytes=64 * 1024 * 1024,
        ),
    )(xp, params, w4)
    out = outp.reshape(steps, N // 2, 2, B).transpose(0, 3, 2, 1)
    return out.reshape(steps, B, N)


def kernel(x, a, b, c, d, bias, w):
    return _snn_forward_t(x, a, b, c, d, bias, w, steps=320, unroll=4,
                          mv_unroll=16)
